# baseline (speedup 1.0000x reference)
import jax
import jax.numpy as jnp
from jax.experimental import pallas as pl

N = 16384
DIM = 256
H = 8
KVF = 4
HKV = 2
DH = 32
BALL = 128
DIMY = 3
NB = N // BALL
SCALE = DH ** -0.5
F32 = jnp.float32


def _softmax(x):
    m = jnp.max(x, axis=-1, keepdims=True)
    e = jnp.exp(x - m)
    return e / jnp.sum(e, axis=-1, keepdims=True)


def _xprime(x_ref, pos_ref, wpe_ref, bpe_ref):
    pos = pos_ref[...]
    rel = pos - jnp.mean(pos, axis=0, keepdims=True)
    return (x_ref[...] + jnp.dot(rel, wpe_ref[...], preferred_element_type=F32)
            + bpe_ref[...])


def _phase1(x_ref, pos_ref, wpe_ref, bpe_ref, wk_ref, wv_ref,
            k_ref, v_ref, ck_ref, cv_ref):
    xp = _xprime(x_ref, pos_ref, wpe_ref, bpe_ref)
    k = jnp.dot(xp, wk_ref[...], preferred_element_type=F32)
    v = jnp.dot(xp, wv_ref[...], preferred_element_type=F32)
    k_ref[...] = k
    v_ref[...] = v
    ck_ref[...] = jnp.mean(k, axis=0, keepdims=True)[None]
    cv_ref[...] = jnp.mean(v, axis=0, keepdims=True)[None]


def _first_argmax(row, iota):
    m = jnp.max(row)
    cand = jnp.where(row == m, iota, NB)
    return jnp.min(cand).astype(jnp.int32)


def _phase2(x_ref, pos_ref, wpe_ref, bpe_ref, wq_ref, wg_ref, wo_ref,
            k_ref, v_ref, ck_ref, cv_ref, o_ref):
    b = pl.program_id(0)
    xp = _xprime(x_ref, pos_ref, wpe_ref, bpe_ref)
    q = jnp.dot(xp, wq_ref[...], preferred_element_type=F32)
    gates = jax.nn.sigmoid(jnp.dot(xp, wg_ref[...], preferred_element_type=F32))
    ckf = ck_ref[...].reshape(NB, HKV * DH)
    cvf = cv_ref[...].reshape(NB, HKV * DH)
    iota = jax.lax.broadcasted_iota(jnp.int32, (1, NB), 1)

    head_outs = [None] * H
    for g in range(HKV):
        c0, c1 = g * DH, (g + 1) * DH
        ckg = ckf[:, c0:c1]
        cvg = cvf[:, c0:c1]
        qfs, attcs = [], []
        imp = jnp.zeros((1, NB), F32)
        for f in range(KVF):
            h = g * KVF + f
            qf = q[:, h * DH:(h + 1) * DH]
            sc = jax.lax.dot_general(qf, ckg, (((1,), (1,)), ((), ())),
                                     preferred_element_type=F32) * SCALE
            attc = _softmax(sc)
            imp = imp + jnp.sum(attc, axis=0, keepdims=True)
            qfs.append(qf)
            attcs.append(attc)
        i0 = _first_argmax(imp, iota)
        imp2 = jnp.where(iota == i0, -jnp.inf, imp)
        i1 = _first_argmax(imp2, iota)
        k0 = k_ref[pl.ds(i0 * BALL, BALL), c0:c1]
        k1 = k_ref[pl.ds(i1 * BALL, BALL), c0:c1]
        v0 = v_ref[pl.ds(i0 * BALL, BALL), c0:c1]
        v1 = v_ref[pl.ds(i1 * BALL, BALL), c0:c1]
        kcat = jnp.concatenate([k0, k1], axis=0)
        vcat = jnp.concatenate([v0, v1], axis=0)
        kl = k_ref[pl.ds(b * BALL, BALL), c0:c1]
        vl = v_ref[pl.ds(b * BALL, BALL), c0:c1]
        for f in range(KVF):
            h = g * KVF + f
            out_c = jnp.dot(attcs[f], cvg, preferred_element_type=F32)
            sf = jax.lax.dot_general(qfs[f], kcat, (((1,), (1,)), ((), ())),
                                     preferred_element_type=F32) * SCALE
            out_f = jnp.dot(_softmax(sf), vcat, preferred_element_type=F32)
            sl = jax.lax.dot_general(qfs[f], kl, (((1,), (1,)), ((), ())),
                                     preferred_element_type=F32) * SCALE
            out_l = jnp.dot(_softmax(sl), vl, preferred_element_type=F32)
            head_outs[h] = (gates[:, h:h + 1] * out_c
                            + gates[:, H + h:H + h + 1] * out_f
                            + gates[:, 2 * H + h:2 * H + h + 1] * out_l)

    out = jnp.concatenate(head_outs, axis=1)
    o_ref[...] = jnp.dot(out, wo_ref[...], preferred_element_type=F32)


def kernel(x, pos, W_pe, b_pe, Wq, Wk, Wv, Wg, Wo):
    bpe = b_pe.reshape(1, DIM)
    full2 = lambda a: pl.BlockSpec(a.shape, lambda b: (0, 0))
    ball2 = lambda w: pl.BlockSpec((BALL, w), lambda b: (b, 0))

    k, v, ck, cv = pl.pallas_call(
        _phase1,
        grid=(NB,),
        in_specs=[ball2(DIM), ball2(DIMY), full2(W_pe), full2(bpe),
                  full2(Wk), full2(Wv)],
        out_specs=[
            pl.BlockSpec((BALL, HKV * DH), lambda b: (b, 0)),
            pl.BlockSpec((BALL, HKV * DH), lambda b: (b, 0)),
            pl.BlockSpec((1, 1, HKV * DH), lambda b: (b, 0, 0)),
            pl.BlockSpec((1, 1, HKV * DH), lambda b: (b, 0, 0)),
        ],
        out_shape=[
            jax.ShapeDtypeStruct((N, HKV * DH), F32),
            jax.ShapeDtypeStruct((N, HKV * DH), F32),
            jax.ShapeDtypeStruct((NB, 1, HKV * DH), F32),
            jax.ShapeDtypeStruct((NB, 1, HKV * DH), F32),
        ],
    )(x, pos, W_pe, bpe, Wk, Wv)

    full3 = pl.BlockSpec((NB, 1, HKV * DH), lambda b: (0, 0, 0))
    out = pl.pallas_call(
        _phase2,
        grid=(NB,),
        in_specs=[ball2(DIM), ball2(DIMY), full2(W_pe), full2(bpe),
                  full2(Wq), full2(Wg), full2(Wo),
                  full2(k), full2(v), full3, full3],
        out_specs=pl.BlockSpec((BALL, DIM), lambda b: (b, 0)),
        out_shape=jax.ShapeDtypeStruct((N, DIM), F32),
    )(x, pos, W_pe, bpe, Wq, Wg, Wo, k, v, ck, cv)
    return out
